# baseline (speedup 1.0000x reference)
import functools

import jax
import jax.numpy as jnp
from jax import lax
from jax.experimental import pallas as pl
from jax.experimental.pallas import tpu as pltpu
from jax.experimental.pallas import tpu_sc as plsc

B = 16384
V = 100000
F_CAT = 26
F_NUM = 13
D = 16

PACK_VB = 33792
PACK_PR = PACK_VB // 8
PACK_NVB = 3
FSTRIDE = PACK_VB * PACK_NVB
PACK_ROWS = FSTRIDE // 8
EMB_ROWS_PAD = F_CAT * FSTRIDE


def _pack_body(x_ref, o_ref):
    for c in range(PACK_VB // 1024):
        x = x_ref[:, c * 1024:(c + 1) * 1024]
        x8 = jnp.concatenate(
            [x[:, s * 128:(s + 1) * 128] for s in range(8)], axis=0)
        o_ref[0, c * 128:(c + 1) * 128, :] = jnp.transpose(x8)


FH = F_CAT // 2


def _make_pack(half):
    return pl.pallas_call(
        _pack_body,
        grid=(FH, PACK_NVB),
        in_specs=[pl.BlockSpec((D, PACK_VB),
                               lambda f, v: (half * FH + f, v))],
        out_specs=pl.BlockSpec((1, PACK_PR, 128), lambda f, v: (f, v, 0)),
        out_shape=jax.ShapeDtypeStruct((FH, PACK_NVB * PACK_PR, 128),
                                       jnp.float32),
    )


_pack_half = (_make_pack(0), _make_pack(1))

NC = 2
NS = 16
NW = NC * NS
HALF_ROWS = (F_CAT // 2) * B
R_PER_W = HALF_ROWS // NW
SUB = 128
NSUB = 13
CHUNK = SUB * NSUB
NCHUNK = R_PER_W // CHUNK
IDX_ROWS_PER_W = R_PER_W // SUB


def _sc_body(gide_hbm, gidl_hbm, emb_tab_hbm, lin_tab_hbm,
             emb_out_hbm, lin_out_hbm,
             idxe_v, idxl_v, rows_v0, rows_v1, lin_v0, lin_v1,
             sem_e0, sem_e1, sem_l0, sem_l1):
    wid = lax.axis_index("s") * NC + lax.axis_index("c")
    base = wid * R_PER_W
    idx_base = wid * IDX_ROWS_PER_W
    pltpu.sync_copy(gide_hbm.at[pl.ds(idx_base, IDX_ROWS_PER_W)], idxe_v)
    pltpu.sync_copy(gidl_hbm.at[pl.ds(idx_base, IDX_ROWS_PER_W)], idxl_v)
    bufs = [(rows_v0, lin_v0, sem_e0, sem_l0),
            (rows_v1, lin_v1, sem_e1, sem_l1)]

    def fire(c):
        rv, lv, se, sl = bufs[c % 2]
        cps = []
        for j in range(NSUB):
            row = c * NSUB + j
            cps.append(pltpu.async_copy(
                emb_tab_hbm.at[idxe_v.at[row]],
                rv.at[pl.ds(j * SUB, SUB)], se))
            cps.append(pltpu.async_copy(
                lin_tab_hbm.at[idxl_v.at[row]],
                lv.at[pl.ds(j * SUB, SUB)], sl))
        return cps

    pend = fire(0)
    for c in range(NCHUNK):
        for cp in pend:
            cp.wait()
        if c + 1 < NCHUNK:
            pend = fire(c + 1)
        rv, lv = bufs[c % 2][0], bufs[c % 2][1]
        pltpu.sync_copy(rv, emb_out_hbm.at[pl.ds(base + c * CHUNK, CHUNK)])
        pltpu.sync_copy(lv, lin_out_hbm.at[pl.ds(base + c * CHUNK, CHUNK)])


_sc_gather = functools.partial(
    pl.kernel,
    out_type=[
        jax.ShapeDtypeStruct((HALF_ROWS, D), jnp.float32),
        jax.ShapeDtypeStruct((HALF_ROWS,), jnp.float32),
    ],
    mesh=plsc.VectorSubcoreMesh(core_axis_name="c", subcore_axis_name="s"),
    scratch_types=[
        pltpu.VMEM((IDX_ROWS_PER_W, SUB), jnp.int32),
        pltpu.VMEM((IDX_ROWS_PER_W, SUB), jnp.int32),
        pltpu.VMEM((CHUNK, D), jnp.float32),
        pltpu.VMEM((CHUNK, D), jnp.float32),
        pltpu.VMEM((CHUNK,), jnp.float32),
        pltpu.VMEM((CHUNK,), jnp.float32),
        pltpu.SemaphoreType.DMA,
        pltpu.SemaphoreType.DMA,
        pltpu.SemaphoreType.DMA,
        pltpu.SemaphoreType.DMA,
    ],
    compiler_params=pltpu.CompilerParams(use_tc_tiling_on_sc=False),
)(_sc_body)

BT = 2048
PRT = BT // 8


def _mlp_body(embp0_ref, embp1_ref, linp0_ref, linp1_ref, nump_ref,
              w1e_ref, w1n_ref, w2_ref, w3_ref,
              w4_ref, cwb_ref, b1_ref, b2_ref, b3_ref, b4_ref, out_ref):
    f32 = jnp.float32
    npk = nump_ref[...]
    acc = jnp.dot(npk, w1n_ref[...], preferred_element_type=f32)
    for f in range(FH):
        acc = acc + jnp.dot(embp0_ref[f], w1e_ref[f],
                            preferred_element_type=f32)
        acc = acc + jnp.dot(embp1_ref[f], w1e_ref[FH + f],
                            preferred_element_type=f32)
    h = jnp.maximum(acc + b1_ref[...], 0.0)
    h = jnp.maximum(jnp.dot(h, w2_ref[...], preferred_element_type=f32)
                    + b2_ref[...], 0.0)
    h = jnp.maximum(jnp.dot(h, w3_ref[...], preferred_element_type=f32)
                    + b3_ref[...], 0.0)
    z = jnp.dot(h, w4_ref[...], preferred_element_type=f32) + b4_ref[...]
    lin_s = linp0_ref[0]
    for f in range(1, FH):
        lin_s = lin_s + linp0_ref[f]
    for f in range(FH):
        lin_s = lin_s + linp1_ref[f]
    z = z + lin_s + jnp.dot(npk, cwb_ref[...], preferred_element_type=f32)
    out_ref[...] = 1.0 / (1.0 + jnp.exp(-z))


def _mlp_call(embp0, embp1, linp0, linp1, nump, w1e, w1n, w2, w3, w4, cwb,
              b1, b2, b3, b4):
    def full(shape):
        return pl.BlockSpec(shape, lambda *_: tuple(0 for _ in shape))

    return pl.pallas_call(
        _mlp_body,
        grid=(B // BT,),
        in_specs=[
            pl.BlockSpec((FH, PRT, 128), lambda i: (0, i, 0)),
            pl.BlockSpec((FH, PRT, 128), lambda i: (0, i, 0)),
            pl.BlockSpec((FH, PRT, 8), lambda i: (0, i, 0)),
            pl.BlockSpec((FH, PRT, 8), lambda i: (0, i, 0)),
            pl.BlockSpec((PRT, 128), lambda i: (i, 0)),
            full((F_CAT, 128, 512)),
            full((128, 512)),
            full((512, 256)),
            full((256, 128)),
            full((128, 8)),
            full((128, 8)),
            full((1, 512)),
            full((1, 256)),
            full((1, 128)),
            full((1, 8)),
        ],
        out_specs=pl.BlockSpec((PRT, 8), lambda i: (i, 0)),
        out_shape=jax.ShapeDtypeStruct((B // 8, 8), jnp.float32),
    )(embp0, embp1, linp0, linp1, nump, w1e, w1n, w2, w3, w4, cwb,
      b1, b2, b3, b4)


def kernel(cat, num, lin_tables, emb_tables, cont_w, W1, b1, W2, b2, W3, b3,
           W4, b4):
    emb_fm = jnp.transpose(emb_tables, (0, 2, 1)).reshape(F_CAT * D, V)
    cat32 = cat.astype(jnp.int32)
    perm = cat32 - (cat32 & 1023) + ((cat32 & 127) << 3) + ((cat32 & 1023) >> 7)
    offs_e = (jnp.arange(FH, dtype=jnp.int32) * FSTRIDE)[:, None]
    offs_l = (jnp.arange(FH, dtype=jnp.int32) * V)[:, None]
    halves = []
    for h in range(2):
        packed = _pack_half[h](emb_fm)
        emb_tab = packed.reshape(FH * FSTRIDE, D)
        lin_flat = lin_tables[h * FH:(h + 1) * FH].reshape(FH * V)
        gidx_e = (perm[h * FH:(h + 1) * FH] + offs_e
                  ).reshape(HALF_ROWS // SUB, SUB)
        gidx_l = (cat32[h * FH:(h + 1) * FH] + offs_l
                  ).reshape(HALF_ROWS // SUB, SUB)
        halves.append(_sc_gather(gidx_e, gidx_l, emb_tab, lin_flat))
    (emb_rows0, lin_rows0), (emb_rows1, lin_rows1) = halves
    f32 = jnp.float32
    eye8 = jnp.eye(8, dtype=f32)
    W1e3 = W1[:F_CAT * D].reshape(F_CAT, D, 64)
    Wb1e = jnp.einsum('jk,fdo->fjdko', eye8, W1e3).reshape(F_CAT, 128, 512)
    W1n = jnp.pad(W1[F_CAT * D:], ((0, 3), (0, 0)))
    Wb1n = jnp.einsum('jk,do->jdko', eye8, W1n).reshape(128, 512)
    Wb2 = jnp.einsum('jk,do->jdko', eye8, W2).reshape(512, 256)
    Wb3 = jnp.einsum('jk,do->jdko', eye8, W3).reshape(256, 128)
    Wb4 = jnp.einsum('jk,do->jdko', eye8, W4).reshape(128, 8)
    cwB = jnp.einsum('jk,c->jck', eye8, jnp.pad(cont_w, (0, 3))
                     ).reshape(128, 8)
    b1t = jnp.tile(b1, 8)[None]
    b2t = jnp.tile(b2, 8)[None]
    b3t = jnp.tile(b3, 8)[None]
    b4t = jnp.tile(b4, 8)[None]
    numP = jnp.pad(num.T, ((0, 0), (0, 3))).reshape(B // 8, 128)
    out8 = _mlp_call(
        emb_rows0.reshape(FH, B // 8, 128),
        emb_rows1.reshape(FH, B // 8, 128),
        lin_rows0.reshape(FH, B // 8, 8),
        lin_rows1.reshape(FH, B // 8, 8),
        numP, Wb1e, Wb1n, Wb2, Wb3, Wb4, cwB, b1t, b2t, b3t, b4t)
    return out8.reshape(B, 1)


# speedup vs baseline: 1.0700x; 1.0700x over previous
import functools

import jax
import jax.numpy as jnp
from jax import lax
from jax.experimental import pallas as pl
from jax.experimental.pallas import tpu as pltpu
from jax.experimental.pallas import tpu_sc as plsc

B = 16384
V = 100000
F_CAT = 26
F_NUM = 13
D = 16

PACK_VB = 33792
PACK_PR = PACK_VB // 8
PACK_NVB = 3
FSTRIDE = PACK_VB * PACK_NVB
PACK_ROWS = FSTRIDE // 8
EMB_ROWS_PAD = F_CAT * FSTRIDE


def _pack_body(x_ref, o_ref):
    for c in range(PACK_VB // 1024):
        x = x_ref[:, c * 1024:(c + 1) * 1024]
        x8 = jnp.concatenate(
            [x[:, s * 128:(s + 1) * 128] for s in range(8)], axis=0)
        o_ref[0, c * 128:(c + 1) * 128, :] = jnp.transpose(x8)


_pack_call = pl.pallas_call(
    _pack_body,
    grid=(F_CAT, PACK_NVB),
    in_specs=[pl.BlockSpec((D, PACK_VB), lambda f, v: (f, v))],
    out_specs=pl.BlockSpec((1, PACK_PR, 128), lambda f, v: (f, v, 0)),
    out_shape=jax.ShapeDtypeStruct((F_CAT, PACK_NVB * PACK_PR, 128),
                                   jnp.float32),
)

NC = 2
NS = 16
NW = NC * NS
TOTAL_ROWS = F_CAT * B
R_PER_W = TOTAL_ROWS // NW
SUB = 128
NSUB = 13
CHUNK = SUB * NSUB
NCHUNK = R_PER_W // CHUNK
IDX_ROWS_PER_W = R_PER_W // SUB


def _sc_body(gide_hbm, gidl_hbm, emb_tab_hbm, lin_tab_hbm,
             emb_out_hbm, lin_out_hbm,
             idxe_v, idxl_v, rows_v0, rows_v1, lin_v0, lin_v1,
             sem_e0, sem_e1, sem_l0, sem_l1):
    wid = lax.axis_index("s") * NC + lax.axis_index("c")
    base = wid * R_PER_W
    idx_base = wid * IDX_ROWS_PER_W
    pltpu.sync_copy(gide_hbm.at[pl.ds(idx_base, IDX_ROWS_PER_W)], idxe_v)
    pltpu.sync_copy(gidl_hbm.at[pl.ds(idx_base, IDX_ROWS_PER_W)], idxl_v)
    bufs = [(rows_v0, lin_v0, sem_e0, sem_l0),
            (rows_v1, lin_v1, sem_e1, sem_l1)]

    def fire(c):
        rv, lv, se, sl = bufs[c % 2]
        cps = []
        for j in range(NSUB):
            row = c * NSUB + j
            cps.append(pltpu.async_copy(
                emb_tab_hbm.at[idxe_v.at[row]],
                rv.at[pl.ds(j * SUB, SUB)], se))
            cps.append(pltpu.async_copy(
                lin_tab_hbm.at[idxl_v.at[row]],
                lv.at[pl.ds(j * SUB, SUB)], sl))
        return cps

    pend = fire(0)
    for c in range(NCHUNK):
        for cp in pend:
            cp.wait()
        if c + 1 < NCHUNK:
            pend = fire(c + 1)
        rv, lv = bufs[c % 2][0], bufs[c % 2][1]
        pltpu.sync_copy(rv, emb_out_hbm.at[pl.ds(base + c * CHUNK, CHUNK)])
        pltpu.sync_copy(lv, lin_out_hbm.at[pl.ds(base + c * CHUNK, CHUNK)])


_sc_gather = functools.partial(
    pl.kernel,
    out_type=[
        jax.ShapeDtypeStruct((TOTAL_ROWS, D), jnp.float32),
        jax.ShapeDtypeStruct((TOTAL_ROWS,), jnp.float32),
    ],
    mesh=plsc.VectorSubcoreMesh(core_axis_name="c", subcore_axis_name="s"),
    scratch_types=[
        pltpu.VMEM((IDX_ROWS_PER_W, SUB), jnp.int32),
        pltpu.VMEM((IDX_ROWS_PER_W, SUB), jnp.int32),
        pltpu.VMEM((CHUNK, D), jnp.float32),
        pltpu.VMEM((CHUNK, D), jnp.float32),
        pltpu.VMEM((CHUNK,), jnp.float32),
        pltpu.VMEM((CHUNK,), jnp.float32),
        pltpu.SemaphoreType.DMA,
        pltpu.SemaphoreType.DMA,
        pltpu.SemaphoreType.DMA,
        pltpu.SemaphoreType.DMA,
    ],
    compiler_params=pltpu.CompilerParams(use_tc_tiling_on_sc=False),
)(_sc_body)

BT = 2048
PRT = BT // 8


def _mlp_body(embp_ref, linp_ref, nump_ref, w1e_ref, w1n_ref, w2_ref, w3_ref,
              w4_ref, cwb_ref, b1_ref, b2_ref, b3_ref, b4_ref, out_ref):
    f32 = jnp.float32
    npk = nump_ref[...]
    acc = jnp.dot(npk, w1n_ref[...], preferred_element_type=f32)
    for f in range(F_CAT):
        acc = acc + jnp.dot(embp_ref[f], w1e_ref[f],
                            preferred_element_type=f32)
    h = jnp.maximum(acc + b1_ref[...], 0.0)
    h = jnp.maximum(jnp.dot(h, w2_ref[...], preferred_element_type=f32)
                    + b2_ref[...], 0.0)
    h = jnp.maximum(jnp.dot(h, w3_ref[...], preferred_element_type=f32)
                    + b3_ref[...], 0.0)
    z = jnp.dot(h, w4_ref[...], preferred_element_type=f32) + b4_ref[...]
    lin_s = linp_ref[0]
    for f in range(1, F_CAT):
        lin_s = lin_s + linp_ref[f]
    z = z + lin_s + jnp.dot(npk, cwb_ref[...], preferred_element_type=f32)
    out_ref[...] = 1.0 / (1.0 + jnp.exp(-z))


def _mlp_call(embp, linp, nump, w1e, w1n, w2, w3, w4, cwb, b1, b2, b3, b4):
    def full(shape):
        return pl.BlockSpec(shape, lambda *_: tuple(0 for _ in shape))

    return pl.pallas_call(
        _mlp_body,
        grid=(B // BT,),
        in_specs=[
            pl.BlockSpec((F_CAT, PRT, 128), lambda i: (0, i, 0)),
            pl.BlockSpec((F_CAT, PRT, 8), lambda i: (0, i, 0)),
            pl.BlockSpec((PRT, 128), lambda i: (i, 0)),
            full((F_CAT, 128, 512)),
            full((128, 512)),
            full((512, 256)),
            full((256, 128)),
            full((128, 8)),
            full((128, 8)),
            full((1, 512)),
            full((1, 256)),
            full((1, 128)),
            full((1, 8)),
        ],
        out_specs=pl.BlockSpec((PRT, 8), lambda i: (i, 0)),
        out_shape=jax.ShapeDtypeStruct((B // 8, 8), jnp.float32),
    )(embp, linp, nump, w1e, w1n, w2, w3, w4, cwb, b1, b2, b3, b4)


def kernel(cat, num, lin_tables, emb_tables, cont_w, W1, b1, W2, b2, W3, b3,
           W4, b4):
    emb_fm = jnp.transpose(emb_tables, (0, 2, 1)).reshape(F_CAT * D, V)
    packed = _pack_call(emb_fm)
    emb_tab = packed.reshape(EMB_ROWS_PAD, D)
    lin_flat = lin_tables.reshape(F_CAT * V)
    cat32 = cat.astype(jnp.int32)
    perm = cat32 - (cat32 & 1023) + ((cat32 & 127) << 3) + ((cat32 & 1023) >> 7)
    gidx_e = (perm + (jnp.arange(F_CAT, dtype=jnp.int32) * FSTRIDE)[:, None]
              ).reshape(TOTAL_ROWS // SUB, SUB)
    gidx_l = (cat32 + (jnp.arange(F_CAT, dtype=jnp.int32) * V)[:, None]
              ).reshape(TOTAL_ROWS // SUB, SUB)
    emb_rows, lin_rows = _sc_gather(gidx_e, gidx_l, emb_tab, lin_flat)
    f32 = jnp.float32
    eye8 = jnp.eye(8, dtype=f32)
    W1e3 = W1[:F_CAT * D].reshape(F_CAT, D, 64)
    Wb1e = jnp.einsum('jk,fdo->fjdko', eye8, W1e3).reshape(F_CAT, 128, 512)
    W1n = jnp.pad(W1[F_CAT * D:], ((0, 3), (0, 0)))
    Wb1n = jnp.einsum('jk,do->jdko', eye8, W1n).reshape(128, 512)
    Wb2 = jnp.einsum('jk,do->jdko', eye8, W2).reshape(512, 256)
    Wb3 = jnp.einsum('jk,do->jdko', eye8, W3).reshape(256, 128)
    Wb4 = jnp.einsum('jk,do->jdko', eye8, W4).reshape(128, 8)
    cwB = jnp.einsum('jk,c->jck', eye8, jnp.pad(cont_w, (0, 3))
                     ).reshape(128, 8)
    b1t = jnp.tile(b1, 8)[None]
    b2t = jnp.tile(b2, 8)[None]
    b3t = jnp.tile(b3, 8)[None]
    b4t = jnp.tile(b4, 8)[None]
    numP = jnp.pad(num.T, ((0, 0), (0, 3))).reshape(B // 8, 128)
    out8 = _mlp_call(
        emb_rows.reshape(F_CAT, B // 8, 128),
        lin_rows.reshape(F_CAT, B // 8, 8),
        numP, Wb1e, Wb1n, Wb2, Wb3, Wb4, cwB, b1t, b2t, b3t, b4t)
    return out8.reshape(B, 1)
